# baseline (speedup 1.0000x reference)
import functools

import jax
import jax.numpy as jnp
from jax import lax
from jax.experimental import pallas as pl
from jax.experimental.pallas import tpu as pltpu
from jax.experimental.pallas import tpu_sc as plsc

NC = 2
NS = 16
NW = NC * NS

Q_LEN = 2048
V_LEN = 2048
D = 32
VOCAB = 257
MAX_POS = (VOCAB - 1) // 2

BT_COLS = 4224
MID = Q_LEN - MAX_POS
SHIFT_COLS = 4096
ROWS_PER_W = Q_LEN // NW


def _shifted_bands(embT):
  COPIES_PER_STEP = 8

  def body(embT_ref, out_ref, bandT):
    g = pl.program_id(0)

    @pl.when(g == 0)
    def _build():
      col0 = embT_ref[:, 0:1]
      bandT[:, 0:MID] = jnp.broadcast_to(col0, (D, MID))
      bandT[:, MID:MID + VOCAB - 1] = embT_ref[:, 0:VOCAB - 1]
      col_last = embT_ref[:, VOCAB - 1:VOCAB]
      bandT[:, MID + VOCAB - 1:BT_COLS] = jnp.broadcast_to(
          col_last, (D, BT_COLS - (MID + VOCAB - 1)))

    for r in range(COPIES_PER_STEP):
      s = g * COPIES_PER_STEP + r
      rolled = pltpu.roll(bandT[:, :], (BT_COLS - s) % BT_COLS, axis=1)
      out_ref[r] = rolled[:, 0:SHIFT_COLS]

  return pl.pallas_call(
      body,
      grid=(128 // COPIES_PER_STEP,),
      in_specs=[pl.BlockSpec((D, VOCAB), lambda g: (0, 0))],
      out_specs=pl.BlockSpec((COPIES_PER_STEP, D, SHIFT_COLS),
                             lambda g: (g, 0, 0)),
      out_shape=jax.ShapeDtypeStruct((128, D, SHIFT_COLS), jnp.float32),
      scratch_shapes=[pltpu.VMEM((D, BT_COLS), jnp.float32)],
  )(embT)


def _band_expand(mt):
  mesh = plsc.VectorSubcoreMesh(core_axis_name="c", subcore_axis_name="s",
                                num_cores=NC)

  CHUNK = V_LEN // 8
  NBUF = 15
  PREF = NBUF - 2
  STEPS = ROWS_PER_W * 8

  @functools.partial(
      pl.kernel,
      mesh=mesh,
      out_type=jax.ShapeDtypeStruct((Q_LEN, D, V_LEN), jnp.float32),
      scratch_types=[
          pltpu.VMEM((NBUF, D, CHUNK), jnp.float32),
          pltpu.SemaphoreType.DMA,
          pltpu.SemaphoreType.DMA,
      ],
  )
  def k(mt_ref, out_hbm, bufs, in_sem, out_sem):
    wid = lax.axis_index("s") * NC + lax.axis_index("c")
    row0 = wid * ROWS_PER_W

    def addr(kstep):
      r = kstep // 8
      h = kstep % 8
      i = row0 + r
      c = Q_LEN - i
      s = lax.rem(c, 128)
      c_al = pl.multiple_of(c - s, 128)
      return i, s, c_al + h * CHUNK, h * CHUNK

    def start_in(kstep):
      i, s, src_c, _ = addr(kstep)
      pltpu.async_copy(
          mt_ref.at[s, :, pl.ds(src_c, CHUNK)],
          bufs.at[kstep % NBUF],
          in_sem,
      )

    def wait_in():
      pltpu.make_async_copy(
          mt_ref.at[0, :, pl.ds(0, CHUNK)], bufs.at[0], in_sem).wait()

    def start_out(kstep):
      i, _, _, dst_c = addr(kstep)
      pltpu.async_copy(
          bufs.at[kstep % NBUF],
          out_hbm.at[i, :, pl.ds(dst_c, CHUNK)],
          out_sem,
      )

    def wait_out():
      pltpu.make_async_copy(
          bufs.at[0], out_hbm.at[row0, :, pl.ds(0, CHUNK)], out_sem).wait()

    for kstep in range(PREF):
      start_in(kstep)

    def body(kstep, carry):
      wait_in()
      start_out(kstep)

      @pl.when(kstep >= NBUF - PREF)
      def _absorb():
        wait_out()

      @pl.when(kstep + PREF < STEPS)
      def _next():
        start_in(kstep + PREF)

      return carry

    lax.fori_loop(0, STEPS, body, 0)
    for _ in range(NBUF - PREF):
      wait_out()

  return k(mt)


def kernel(q, v, embeddings):
  del q, v
  out = _band_expand(_shifted_bands(embeddings.T))
  return jnp.transpose(out, (0, 2, 1))


# speedup vs baseline: 1.0061x; 1.0061x over previous
import functools

import jax
import jax.numpy as jnp
from jax import lax
from jax.experimental import pallas as pl
from jax.experimental.pallas import tpu as pltpu
from jax.experimental.pallas import tpu_sc as plsc

NC = 2
NS = 16
NW = NC * NS

Q_LEN = 2048
V_LEN = 2048
D = 32
VOCAB = 257
MAX_POS = (VOCAB - 1) // 2

BT_COLS = 4224
MID = Q_LEN - MAX_POS
SHIFT_COLS = 4096
ROWS_PER_W = Q_LEN // NW


def _shifted_bands(embT):
  COPIES_PER_STEP = 16

  def body(embT_ref, out_ref, bandT):
    g = pl.program_id(0)

    @pl.when(g == 0)
    def _build():
      col0 = embT_ref[:, 0:1]
      bandT[:, 0:MID] = jnp.broadcast_to(col0, (D, MID))
      bandT[:, MID:MID + VOCAB - 1] = embT_ref[:, 0:VOCAB - 1]
      col_last = embT_ref[:, VOCAB - 1:VOCAB]
      bandT[:, MID + VOCAB - 1:BT_COLS] = jnp.broadcast_to(
          col_last, (D, BT_COLS - (MID + VOCAB - 1)))

    for r in range(COPIES_PER_STEP):
      s = g * COPIES_PER_STEP + r
      rolled = pltpu.roll(bandT[:, :], (BT_COLS - s) % BT_COLS, axis=1)
      out_ref[r] = rolled[:, 0:SHIFT_COLS]

  return pl.pallas_call(
      body,
      grid=(128 // COPIES_PER_STEP,),
      in_specs=[pl.BlockSpec((D, VOCAB), lambda g: (0, 0))],
      out_specs=pl.BlockSpec((COPIES_PER_STEP, D, SHIFT_COLS),
                             lambda g: (g, 0, 0)),
      out_shape=jax.ShapeDtypeStruct((128, D, SHIFT_COLS), jnp.float32),
      scratch_shapes=[pltpu.VMEM((D, BT_COLS), jnp.float32)],
  )(embT)


def _band_expand(mt):
  mesh = plsc.VectorSubcoreMesh(core_axis_name="c", subcore_axis_name="s",
                                num_cores=NC)

  CHUNK = V_LEN // 4
  NBUF = 7
  PREF = NBUF - 2
  STEPS = ROWS_PER_W * 4

  @functools.partial(
      pl.kernel,
      mesh=mesh,
      out_type=jax.ShapeDtypeStruct((Q_LEN, D, V_LEN), jnp.float32),
      scratch_types=[
          pltpu.VMEM((NBUF, D, CHUNK), jnp.float32),
          pltpu.SemaphoreType.DMA,
          pltpu.SemaphoreType.DMA,
      ],
  )
  def k(mt_ref, out_hbm, bufs, in_sem, out_sem):
    wid = lax.axis_index("s") * NC + lax.axis_index("c")
    row0 = wid * ROWS_PER_W

    def addr(kstep):
      r = kstep // 4
      h = kstep % 4
      i = row0 + r
      c = Q_LEN - i
      s = lax.rem(c, 128)
      c_al = pl.multiple_of(c - s, 128)
      return i, s, c_al + h * CHUNK, h * CHUNK

    def start_in(kstep):
      i, s, src_c, _ = addr(kstep)
      pltpu.async_copy(
          mt_ref.at[s, :, pl.ds(src_c, CHUNK)],
          bufs.at[kstep % NBUF],
          in_sem,
      )

    def wait_in():
      pltpu.make_async_copy(
          mt_ref.at[0, :, pl.ds(0, CHUNK)], bufs.at[0], in_sem).wait()

    def start_out(kstep):
      i, _, _, dst_c = addr(kstep)
      pltpu.async_copy(
          bufs.at[kstep % NBUF],
          out_hbm.at[i, :, pl.ds(dst_c, CHUNK)],
          out_sem,
      )

    def wait_out():
      pltpu.make_async_copy(
          bufs.at[0], out_hbm.at[row0, :, pl.ds(0, CHUNK)], out_sem).wait()

    for kstep in range(PREF):
      start_in(kstep)

    def body(kstep, carry):
      wait_in()
      start_out(kstep)

      @pl.when(kstep >= NBUF - PREF)
      def _absorb():
        wait_out()

      @pl.when(kstep + PREF < STEPS)
      def _next():
        start_in(kstep + PREF)

      return carry

    lax.fori_loop(0, STEPS, body, 0)
    for _ in range(NBUF - PREF):
      wait_out()

  return k(mt)


def kernel(q, v, embeddings):
  del q, v
  out = _band_expand(_shifted_bands(embeddings.T))
  return jnp.transpose(out, (0, 2, 1))
